# Initial kernel scaffold; baseline (speedup 1.0000x reference)
import functools

import jax
import jax.numpy as jnp
from jax import lax
from jax.experimental import pallas as pl
from jax.experimental.pallas import tpu as pltpu
from jax.experimental.pallas import tpu_sc as plsc

N_NODES = 10000
N_EDGES = 320000
D = 128

NC = 2
NS = 16
NW = NC * NS

EDGES_PER_TILE = N_EDGES // NW
CHUNK = 80
N_CHUNKS = EDGES_PER_TILE // CHUNK
ROWS_PER_TILE = N_NODES // NS


def _sc_aggregate(features, src, tgt):
    mesh = plsc.VectorSubcoreMesh(core_axis_name="c", subcore_axis_name="s")

    @functools.partial(
        pl.kernel,
        mesh=mesh,
        out_type=jax.ShapeDtypeStruct((NC, N_NODES, D), jnp.float32),
        scratch_types=[
            pltpu.VMEM((CHUNK,), jnp.int32),
            pltpu.VMEM((CHUNK,), jnp.int32),
            pltpu.VMEM((CHUNK, D), jnp.float32),
            pltpu.VMEM((ROWS_PER_TILE, D), jnp.float32),
            pltpu.VMEM_SHARED((N_NODES, D), jnp.float32),
            pltpu.SemaphoreType.DMA,
        ],
    )
    def agg_kernel(feat_hbm, src_hbm, tgt_hbm, out_hbm,
                   idx_t, idx_s, rows, zbuf, agg_sh, sem):
        c = lax.axis_index("c")
        s = lax.axis_index("s")

        def zloop(i, carry):
            zbuf[i // 8, pl.ds((i % 8) * 16, 16)] = jnp.zeros((16,), jnp.float32)
            return carry
        lax.fori_loop(0, ROWS_PER_TILE * 8, zloop, 0)
        pltpu.sync_copy(zbuf, agg_sh.at[pl.ds(s * ROWS_PER_TILE, ROWS_PER_TILE)])
        plsc.subcore_barrier()

        base_e = (c * NS + s) * EDGES_PER_TILE

        def eloop(i, carry):
            off = base_e + i * CHUNK
            pltpu.sync_copy(tgt_hbm.at[pl.ds(off, CHUNK)], idx_t)
            pltpu.async_copy(feat_hbm.at[idx_t], rows, sem).wait()
            pltpu.sync_copy(src_hbm.at[pl.ds(off, CHUNK)], idx_s)
            pltpu.sync_copy(rows, agg_sh.at[idx_s], add=True)
            return carry
        lax.fori_loop(0, N_CHUNKS, eloop, 0)

        plsc.subcore_barrier()

        r0 = s * ROWS_PER_TILE
        pltpu.sync_copy(agg_sh.at[pl.ds(r0, ROWS_PER_TILE)], zbuf)
        pltpu.sync_copy(zbuf, out_hbm.at[c, pl.ds(r0, ROWS_PER_TILE)])

    return agg_kernel(features, src, tgt)


def _tc_dense(features, agg0, agg1, W1, W2, b, gamma, beta):
    def body(x_ref, a0_ref, a1_ref, w1_ref, w2_ref, b_ref, g_ref, be_ref, o_ref):
        x = x_ref[...]
        a = a0_ref[...] + a1_ref[...]
        dn = (((1,), (1,)), ((), ()))
        y = lax.dot_general(x, w1_ref[...], dn, preferred_element_type=jnp.float32)
        y = y + lax.dot_general(a, w2_ref[...], dn, preferred_element_type=jnp.float32)
        y = jnp.maximum(y + b_ref[...], 0.0)
        inv_n = 1.0 / N_NODES
        mean = jnp.sum(y, axis=0, keepdims=True) * inv_n
        var = jnp.sum(y * y, axis=0, keepdims=True) * inv_n - mean * mean
        scale = g_ref[...] / jnp.sqrt(var + 1e-5)
        shift = be_ref[...] - mean * scale
        z = y * scale + shift
        rn = jnp.sqrt(jnp.sum(z * z, axis=1, keepdims=True))
        o_ref[...] = z / (rn + 1e-6)

    return pl.pallas_call(
        body,
        out_shape=jax.ShapeDtypeStruct((N_NODES, D), jnp.float32),
    )(features, agg0, agg1, W1, W2, b, gamma, beta)


def kernel(features, edge_index, W, b, gamma, beta):
    ei = edge_index.astype(jnp.int32)
    src = ei[0]
    tgt = ei[1]
    agg = _sc_aggregate(features, src, tgt)
    W1 = W[:, :D]
    W2 = W[:, D:]
    out = _tc_dense(features, agg[0], agg[1], W1, W2,
                    b.reshape(1, D), gamma.reshape(1, D), beta.reshape(1, D))
    return out


# speedup vs baseline: 2.9963x; 2.9963x over previous
import functools

import jax
import jax.numpy as jnp
from jax import lax
from jax.experimental import pallas as pl
from jax.experimental.pallas import tpu as pltpu
from jax.experimental.pallas import tpu_sc as plsc

N_NODES = 10000
N_EDGES = 320000
D = 128

NC = 2
NS = 16
NW = NC * NS

EDGES_PER_TILE = N_EDGES // NS
CHUNK = 80
N_CHUNKS = EDGES_PER_TILE // CHUNK

ROWS_PER_CORE = N_NODES // NC
DUMP_ROWS = 64
ACC_ROWS = ROWS_PER_CORE + DUMP_ROWS
ROWS_PER_TILE = 312
TAIL_ROWS = ROWS_PER_CORE - NS * ROWS_PER_TILE
TAIL_OFF = NS * ROWS_PER_TILE


def _sc_aggregate(features, src, tgt):
    mesh = plsc.VectorSubcoreMesh(core_axis_name="c", subcore_axis_name="s")

    @functools.partial(
        pl.kernel,
        mesh=mesh,
        out_type=jax.ShapeDtypeStruct((N_NODES, D), jnp.float32),
        scratch_types=[
            pltpu.VMEM((CHUNK,), jnp.int32),
            pltpu.VMEM((CHUNK,), jnp.int32),
            pltpu.VMEM((CHUNK, D), jnp.float32),
            pltpu.VMEM((ROWS_PER_TILE, D), jnp.float32),
            pltpu.VMEM_SHARED((ACC_ROWS, D), jnp.float32),
            pltpu.SemaphoreType.DMA,
        ],
    )
    def agg_kernel(feat_hbm, src_hbm, tgt_hbm, out_hbm,
                   idx_t, idx_s, rows, zbuf, agg_sh, sem):
        c = lax.axis_index("c")
        s = lax.axis_index("s")
        row_base = c * ROWS_PER_CORE

        def zloop(i, carry):
            zbuf[i // 8, pl.ds((i % 8) * 16, 16)] = jnp.zeros((16,), jnp.float32)
            return carry
        lax.fori_loop(0, ROWS_PER_TILE * 8, zloop, 0)
        pltpu.sync_copy(zbuf, agg_sh.at[pl.ds(s * ROWS_PER_TILE, ROWS_PER_TILE)])

        @pl.when(s == 0)
        def _zero_tail():
            pltpu.sync_copy(zbuf.at[pl.ds(0, TAIL_ROWS)],
                            agg_sh.at[pl.ds(TAIL_OFF, TAIL_ROWS)])

        plsc.subcore_barrier()

        base_e = s * EDGES_PER_TILE

        def eloop(i, carry):
            off = base_e + i * CHUNK
            pltpu.sync_copy(tgt_hbm.at[pl.ds(off, CHUNK)], idx_t)
            pltpu.async_copy(feat_hbm.at[idx_t], rows, sem).wait()
            pltpu.sync_copy(src_hbm.at[pl.ds(off, CHUNK)], idx_s)
            for j in range(CHUNK // 16):
                v = idx_s[pl.ds(j * 16, 16)]
                local = v - row_base
                inb = jnp.logical_and(local >= 0, local < ROWS_PER_CORE)
                dump = ROWS_PER_CORE + jnp.bitwise_and(v, DUMP_ROWS - 1)
                idx_s[pl.ds(j * 16, 16)] = jnp.where(inb, local, dump)
            pltpu.sync_copy(rows, agg_sh.at[idx_s], add=True)
            return carry
        lax.fori_loop(0, N_CHUNKS, eloop, 0)

        plsc.subcore_barrier()

        r0 = s * ROWS_PER_TILE
        pltpu.sync_copy(agg_sh.at[pl.ds(r0, ROWS_PER_TILE)], zbuf)
        pltpu.sync_copy(zbuf, out_hbm.at[pl.ds(row_base + r0, ROWS_PER_TILE)])

        @pl.when(s == 0)
        def _copy_tail():
            pltpu.sync_copy(agg_sh.at[pl.ds(TAIL_OFF, TAIL_ROWS)],
                            zbuf.at[pl.ds(0, TAIL_ROWS)])
            pltpu.sync_copy(zbuf.at[pl.ds(0, TAIL_ROWS)],
                            out_hbm.at[pl.ds(row_base + TAIL_OFF, TAIL_ROWS)])

    return agg_kernel(features, src, tgt)


def _tc_dense(features, agg, W1, W2, b, gamma, beta):
    def body(x_ref, a_ref, w1_ref, w2_ref, b_ref, g_ref, be_ref, o_ref):
        x = x_ref[...]
        a = a_ref[...]
        dn = (((1,), (1,)), ((), ()))
        y = lax.dot_general(x, w1_ref[...], dn, preferred_element_type=jnp.float32)
        y = y + lax.dot_general(a, w2_ref[...], dn, preferred_element_type=jnp.float32)
        y = jnp.maximum(y + b_ref[...], 0.0)
        inv_n = 1.0 / N_NODES
        mean = jnp.sum(y, axis=0, keepdims=True) * inv_n
        var = jnp.sum(y * y, axis=0, keepdims=True) * inv_n - mean * mean
        scale = g_ref[...] / jnp.sqrt(var + 1e-5)
        shift = be_ref[...] - mean * scale
        z = y * scale + shift
        rn = jnp.sqrt(jnp.sum(z * z, axis=1, keepdims=True))
        o_ref[...] = z / (rn + 1e-6)

    return pl.pallas_call(
        body,
        out_shape=jax.ShapeDtypeStruct((N_NODES, D), jnp.float32),
    )(features, agg, W1, W2, b, gamma, beta)


def kernel(features, edge_index, W, b, gamma, beta):
    ei = edge_index.astype(jnp.int32)
    src = ei[0]
    tgt = ei[1]
    agg = _sc_aggregate(features, src, tgt)
    W1 = W[:, :D]
    W2 = W[:, D:]
    out = _tc_dense(features, agg, W1, W2,
                    b.reshape(1, D), gamma.reshape(1, D), beta.reshape(1, D))
    return out


# speedup vs baseline: 5.2722x; 1.7596x over previous
import functools

import jax
import jax.numpy as jnp
from jax import lax
from jax.experimental import pallas as pl
from jax.experimental.pallas import tpu as pltpu
from jax.experimental.pallas import tpu_sc as plsc

N_NODES = 10000
N_EDGES = 320000
D = 128

NC = 2
NS = 16
NW = NC * NS

EDGES_PER_TILE = N_EDGES // NS
CHUNK = 80
N_CHUNKS = EDGES_PER_TILE // CHUNK

ROWS_PER_CORE = N_NODES // NC
DUMP_ROWS = 64
ACC_ROWS = ROWS_PER_CORE + DUMP_ROWS
ROWS_PER_TILE = 312
TAIL_ROWS = ROWS_PER_CORE - NS * ROWS_PER_TILE
TAIL_OFF = NS * ROWS_PER_TILE


def _sc_aggregate(features, src, tgt):
    mesh = plsc.VectorSubcoreMesh(core_axis_name="c", subcore_axis_name="s")

    @functools.partial(
        pl.kernel,
        mesh=mesh,
        out_type=jax.ShapeDtypeStruct((N_NODES, D), jnp.float32),
        scratch_types=[
            pltpu.VMEM((CHUNK,), jnp.int32),
            pltpu.VMEM((CHUNK,), jnp.int32),
            pltpu.VMEM((CHUNK,), jnp.int32),
            pltpu.VMEM((CHUNK,), jnp.int32),
            pltpu.VMEM((CHUNK, D), jnp.float32),
            pltpu.VMEM((CHUNK, D), jnp.float32),
            pltpu.VMEM((ROWS_PER_TILE, D), jnp.float32),
            pltpu.VMEM_SHARED((ACC_ROWS, D), jnp.float32),
            pltpu.SemaphoreType.DMA,
            pltpu.SemaphoreType.DMA,
            pltpu.SemaphoreType.DMA,
            pltpu.SemaphoreType.DMA,
        ],
    )
    def agg_kernel(feat_hbm, src_hbm, tgt_hbm, out_hbm,
                   idx_tA, idx_tB, idx_sA, idx_sB, rowsA, rowsB,
                   zbuf, agg_sh, gsemA, gsemB, ssemA, ssemB):
        c = lax.axis_index("c")
        s = lax.axis_index("s")
        row_base = c * ROWS_PER_CORE

        def zloop(i, carry):
            zbuf[i // 8, pl.ds((i % 8) * 16, 16)] = jnp.zeros((16,), jnp.float32)
            return carry
        lax.fori_loop(0, ROWS_PER_TILE * 8, zloop, 0)
        pltpu.sync_copy(zbuf, agg_sh.at[pl.ds(s * ROWS_PER_TILE, ROWS_PER_TILE)])

        @pl.when(s == 0)
        def _zero_tail():
            pltpu.sync_copy(zbuf.at[pl.ds(0, TAIL_ROWS)],
                            agg_sh.at[pl.ds(TAIL_OFF, TAIL_ROWS)])

        plsc.subcore_barrier()

        base_e = s * EDGES_PER_TILE

        def remap(idx_s):
            for j in range(CHUNK // 16):
                v = idx_s[pl.ds(j * 16, 16)]
                local = v - row_base
                inb = jnp.logical_and(local >= 0, local < ROWS_PER_CORE)
                dump = ROWS_PER_CORE + jnp.bitwise_and(v, DUMP_ROWS - 1)
                idx_s[pl.ds(j * 16, 16)] = jnp.where(inb, local, dump)

        def eloop(k, carry):
            offA = base_e + (2 * k) * CHUNK
            offB = offA + CHUNK

            @pl.when(k > 0)
            def _drainA():
                pltpu.make_async_copy(rowsA, agg_sh.at[idx_sA], ssemA).wait()
            pltpu.sync_copy(tgt_hbm.at[pl.ds(offA, CHUNK)], idx_tA)
            cpA = pltpu.async_copy(feat_hbm.at[idx_tA], rowsA, gsemA)

            @pl.when(k > 0)
            def _drainB():
                pltpu.make_async_copy(rowsB, agg_sh.at[idx_sB], ssemB).wait()
            pltpu.sync_copy(tgt_hbm.at[pl.ds(offB, CHUNK)], idx_tB)
            cpB = pltpu.async_copy(feat_hbm.at[idx_tB], rowsB, gsemB)

            pltpu.sync_copy(src_hbm.at[pl.ds(offA, CHUNK)], idx_sA)
            remap(idx_sA)
            pltpu.sync_copy(src_hbm.at[pl.ds(offB, CHUNK)], idx_sB)
            remap(idx_sB)

            cpA.wait()
            pltpu.async_copy(rowsA, agg_sh.at[idx_sA], ssemA, add=True)
            cpB.wait()
            pltpu.async_copy(rowsB, agg_sh.at[idx_sB], ssemB, add=True)
            return carry
        lax.fori_loop(0, N_CHUNKS // 2, eloop, 0)
        pltpu.make_async_copy(rowsA, agg_sh.at[idx_sA], ssemA).wait()
        pltpu.make_async_copy(rowsB, agg_sh.at[idx_sB], ssemB).wait()

        plsc.subcore_barrier()

        r0 = s * ROWS_PER_TILE
        pltpu.sync_copy(agg_sh.at[pl.ds(r0, ROWS_PER_TILE)], zbuf)
        pltpu.sync_copy(zbuf, out_hbm.at[pl.ds(row_base + r0, ROWS_PER_TILE)])

        @pl.when(s == 0)
        def _copy_tail():
            pltpu.sync_copy(agg_sh.at[pl.ds(TAIL_OFF, TAIL_ROWS)],
                            zbuf.at[pl.ds(0, TAIL_ROWS)])
            pltpu.sync_copy(zbuf.at[pl.ds(0, TAIL_ROWS)],
                            out_hbm.at[pl.ds(row_base + TAIL_OFF, TAIL_ROWS)])

    return agg_kernel(features, src, tgt)


def _tc_dense(features, agg, W1, W2, b, gamma, beta):
    def body(x_ref, a_ref, w1_ref, w2_ref, b_ref, g_ref, be_ref, o_ref):
        x = x_ref[...]
        a = a_ref[...]
        dn = (((1,), (1,)), ((), ()))
        y = lax.dot_general(x, w1_ref[...], dn, preferred_element_type=jnp.float32)
        y = y + lax.dot_general(a, w2_ref[...], dn, preferred_element_type=jnp.float32)
        y = jnp.maximum(y + b_ref[...], 0.0)
        inv_n = 1.0 / N_NODES
        mean = jnp.sum(y, axis=0, keepdims=True) * inv_n
        var = jnp.sum(y * y, axis=0, keepdims=True) * inv_n - mean * mean
        scale = g_ref[...] / jnp.sqrt(var + 1e-5)
        shift = be_ref[...] - mean * scale
        z = y * scale + shift
        rn = jnp.sqrt(jnp.sum(z * z, axis=1, keepdims=True))
        o_ref[...] = z / (rn + 1e-6)

    return pl.pallas_call(
        body,
        out_shape=jax.ShapeDtypeStruct((N_NODES, D), jnp.float32),
    )(features, agg, W1, W2, b, gamma, beta)


def kernel(features, edge_index, W, b, gamma, beta):
    ei = edge_index.astype(jnp.int32)
    src = ei[0]
    tgt = ei[1]
    agg = _sc_aggregate(features, src, tgt)
    W1 = W[:, :D]
    W2 = W[:, D:]
    out = _tc_dense(features, agg, W1, W2,
                    b.reshape(1, D), gamma.reshape(1, D), beta.reshape(1, D))
    return out


# speedup vs baseline: 9.0518x; 1.7169x over previous
import functools

import jax
import jax.numpy as jnp
from jax import lax
from jax.experimental import pallas as pl
from jax.experimental.pallas import tpu as pltpu
from jax.experimental.pallas import tpu_sc as plsc

N_NODES = 10000
N_EDGES = 320000
D = 128

NC = 2
NS = 16
NW = NC * NS

EDGES_PER_TILE = N_EDGES // NS
CHUNK = 80
N_CHUNKS = EDGES_PER_TILE // CHUNK
NBUF = 5
LOOK_G = 2
LOOK_I = 4

ROWS_PER_CORE = N_NODES // NC
DUMP_ROWS = 64
ACC_ROWS = ROWS_PER_CORE + DUMP_ROWS
ROWS_PER_TILE = 312
ZROWS = 104
TAIL_ROWS = ROWS_PER_CORE - NS * ROWS_PER_TILE
TAIL_OFF = NS * ROWS_PER_TILE


def _sc_aggregate(features, src, tgt):
    mesh = plsc.VectorSubcoreMesh(core_axis_name="c", subcore_axis_name="s")

    @functools.partial(
        pl.kernel,
        mesh=mesh,
        out_type=jax.ShapeDtypeStruct((N_NODES, D), jnp.float32),
        scratch_types=(
            [pltpu.VMEM((CHUNK,), jnp.int32) for _ in range(NBUF)]
            + [pltpu.VMEM((CHUNK,), jnp.int32) for _ in range(NBUF)]
            + [pltpu.VMEM((CHUNK,), jnp.int32) for _ in range(NBUF)]
            + [pltpu.VMEM((CHUNK, D), jnp.float32) for _ in range(NBUF)]
            + [pltpu.VMEM((ZROWS, D), jnp.float32)]
            + [pltpu.VMEM_SHARED((ACC_ROWS, D), jnp.float32)]
            + [pltpu.SemaphoreType.DMA] * (4 * NBUF)
        ),
    )
    def agg_kernel(feat_hbm, src_hbm, tgt_hbm, out_hbm, *scratch):
        idx_t = scratch[0:NBUF]
        idx_r = scratch[NBUF:2 * NBUF]
        idx_s = scratch[2 * NBUF:3 * NBUF]
        rows = scratch[3 * NBUF:4 * NBUF]
        zbuf = scratch[4 * NBUF]
        agg_sh = scratch[4 * NBUF + 1]
        sems = scratch[4 * NBUF + 2:]
        isem_t = sems[0:NBUF]
        isem_s = sems[NBUF:2 * NBUF]
        gsem = sems[2 * NBUF:3 * NBUF]
        ssem = sems[3 * NBUF:4 * NBUF]

        c = lax.axis_index("c")
        s = lax.axis_index("s")
        row_base = c * ROWS_PER_CORE
        base_e = s * EDGES_PER_TILE

        def zloop(i, carry):
            zbuf[i // 8, pl.ds((i % 8) * 16, 16)] = jnp.zeros((16,), jnp.float32)
            return carry
        lax.fori_loop(0, ZROWS * 8, zloop, 0)
        r0 = s * ROWS_PER_TILE
        for p in range(ROWS_PER_TILE // ZROWS):
            pltpu.sync_copy(zbuf, agg_sh.at[pl.ds(r0 + p * ZROWS, ZROWS)])

        @pl.when(s == 0)
        def _zero_tail():
            pltpu.sync_copy(zbuf.at[pl.ds(0, TAIL_ROWS)],
                            agg_sh.at[pl.ds(TAIL_OFF, TAIL_ROWS)])

        plsc.subcore_barrier()

        def fire_idx(g, q):
            off = base_e + g * CHUNK
            pltpu.async_copy(tgt_hbm.at[pl.ds(off, CHUNK)], idx_t[q], isem_t[q])
            pltpu.async_copy(src_hbm.at[pl.ds(off, CHUNK)], idx_r[q], isem_s[q])

        def fire_gather(g, q):
            pltpu.make_async_copy(tgt_hbm.at[pl.ds(0, CHUNK)], idx_t[q],
                                  isem_t[q]).wait()
            pltpu.async_copy(feat_hbm.at[idx_t[q]], rows[q], gsem[q])
            pltpu.make_async_copy(src_hbm.at[pl.ds(0, CHUNK)], idx_r[q],
                                  isem_s[q]).wait()
            for j in range(CHUNK // 16):
                v = idx_r[q][pl.ds(j * 16, 16)]
                local = v - row_base
                inb = jnp.logical_and(local >= 0, local < ROWS_PER_CORE)
                dump = ROWS_PER_CORE + jnp.bitwise_and(v, DUMP_ROWS - 1)
                idx_s[q][pl.ds(j * 16, 16)] = jnp.where(inb, local, dump)

        def drain_scatter(q):
            pltpu.make_async_copy(rows[q], agg_sh.at[idx_s[q]], ssem[q]).wait()

        for g in range(LOOK_I):
            fire_idx(g, g % NBUF)
        for g in range(LOOK_G):
            fire_gather(g, g % NBUF)

        def step(gc, j):
            g4 = gc + LOOK_I
            q4 = (j + LOOK_I) % NBUF
            g2 = gc + LOOK_G
            q2 = (j + LOOK_G) % NBUF

            @pl.when(g4 < N_CHUNKS)
            def _prefetch_idx():
                fire_idx(g4, q4)

            @pl.when(g2 < N_CHUNKS)
            def _prefetch_rows():
                @pl.when(g2 >= NBUF)
                def _drain():
                    drain_scatter(q2)
                fire_gather(g2, q2)

            pltpu.make_async_copy(feat_hbm.at[idx_t[j]], rows[j],
                                  gsem[j]).wait()
            pltpu.async_copy(rows[j], agg_sh.at[idx_s[j]], ssem[j], add=True)

        def eloop(m, carry):
            for j in range(NBUF):
                step(m * NBUF + j, j)
            return carry
        lax.fori_loop(0, N_CHUNKS // NBUF, eloop, 0)
        for q in range(NBUF):
            drain_scatter(q)

        plsc.subcore_barrier()

        for p in range(ROWS_PER_TILE // ZROWS):
            pltpu.sync_copy(agg_sh.at[pl.ds(r0 + p * ZROWS, ZROWS)], zbuf)
            pltpu.sync_copy(zbuf, out_hbm.at[pl.ds(row_base + r0 + p * ZROWS, ZROWS)])

        @pl.when(s == 0)
        def _copy_tail():
            pltpu.sync_copy(agg_sh.at[pl.ds(TAIL_OFF, TAIL_ROWS)],
                            zbuf.at[pl.ds(0, TAIL_ROWS)])
            pltpu.sync_copy(zbuf.at[pl.ds(0, TAIL_ROWS)],
                            out_hbm.at[pl.ds(row_base + TAIL_OFF, TAIL_ROWS)])

    return agg_kernel(features, src, tgt)


def _tc_dense(features, agg, W1, W2, b, gamma, beta):
    def body(x_ref, a_ref, w1_ref, w2_ref, b_ref, g_ref, be_ref, o_ref):
        x = x_ref[...]
        a = a_ref[...]
        dn = (((1,), (1,)), ((), ()))
        y = lax.dot_general(x, w1_ref[...], dn, preferred_element_type=jnp.float32)
        y = y + lax.dot_general(a, w2_ref[...], dn, preferred_element_type=jnp.float32)
        y = jnp.maximum(y + b_ref[...], 0.0)
        inv_n = 1.0 / N_NODES
        mean = jnp.sum(y, axis=0, keepdims=True) * inv_n
        var = jnp.sum(y * y, axis=0, keepdims=True) * inv_n - mean * mean
        scale = g_ref[...] / jnp.sqrt(var + 1e-5)
        shift = be_ref[...] - mean * scale
        z = y * scale + shift
        rn = jnp.sqrt(jnp.sum(z * z, axis=1, keepdims=True))
        o_ref[...] = z / (rn + 1e-6)

    return pl.pallas_call(
        body,
        out_shape=jax.ShapeDtypeStruct((N_NODES, D), jnp.float32),
    )(features, agg, W1, W2, b, gamma, beta)


def kernel(features, edge_index, W, b, gamma, beta):
    ei = edge_index.astype(jnp.int32)
    src = ei[0]
    tgt = ei[1]
    agg = _sc_aggregate(features, src, tgt)
    W1 = W[:, :D]
    W2 = W[:, D:]
    out = _tc_dense(features, agg, W1, W2,
                    b.reshape(1, D), gamma.reshape(1, D), beta.reshape(1, D))
    return out
